# speedup vs baseline: 4.0277x; 4.0277x over previous
import functools

import jax
import jax.numpy as jnp
from jax import lax
from jax.experimental import pallas as pl
from jax.experimental.pallas import tpu as pltpu

MASK_VAL = -100000000.0


def _leaky(x):
    return jnp.where(x >= 0, x, 0.01 * x)


def _tables_kernel(snapshots_ref, w_dst_t_ref, b_dst_ref, types_ref,
                   snap_emb_ref, w_dstt_t_ref, b_dstt_ref,
                   ea_ref, eb_ref, w_srct_t_ref, b_srct_ref,
                   attn_t_ref, bias_t_ref,
                   feat_dst_ref, et_tab_ref):
    f32 = jnp.float32
    feat_dst = jnp.dot(snapshots_ref[...], w_dst_t_ref[...],
                       preferred_element_type=f32) + b_dst_ref[...]
    feat_dst_ref[...] = feat_dst

    ncols = snap_emb_ref.shape[0]
    iota = lax.broadcasted_iota(jnp.int32, (1, ncols), 1)
    onehot = (types_ref[...] == iota).astype(f32)
    dst_type_h = jnp.dot(onehot, snap_emb_ref[...], preferred_element_type=f32)
    dproj = jnp.dot(dst_type_h, w_dstt_t_ref[...],
                    preferred_element_type=f32) + b_dstt_ref[...]

    ea = ea_ref[...]
    eb = eb_ref[...]
    attn_t = attn_t_ref[...]
    bias_t = bias_t_ref[...]
    cols = []
    for a in (0, 1):
        for b in (0, 1):
            t = ea[a:a + 1, :] + eb[b:b + 1, :]
            tp = jnp.dot(t, w_srct_t_ref[...],
                         preferred_element_type=f32) + b_srct_ref[...]
            col = jnp.sum(_leaky(dproj + tp + bias_t) * attn_t,
                          axis=1, keepdims=True)
            cols.append(col)
    et_tab_ref[...] = jnp.concatenate(cols, axis=1)


def _main_kernel(base_ref, maxs_ref,
                 papers_ref, seg_ref, a_ref, b_ref,
                 feat_dst_ref, et_tab_ref, w_src_t_ref, bias_eff_ref,
                 attn_ref, w_out_t_ref, b_out_ref,
                 out_ref, acc_ref, d_ref, fd_ref,
                 *, grid_n, win, nseg):
    f32 = jnp.float32
    i = pl.program_id(0)

    @pl.when(i == 0)
    def _init():
        acc_ref[...] = jnp.zeros_like(acc_ref)
        d_ref[...] = jnp.zeros_like(d_ref)

    papers = papers_ref[...]
    seg = seg_ref[0]
    a = a_ref[0]
    b = b_ref[0]

    base = base_ref[i]
    base_al = (base // 8) * 8
    nch = (maxs_ref[i] - base_al) // win + 1

    iota_w = lax.broadcasted_iota(jnp.int32, (1, win), 1)
    chunk_of_row = (seg - base_al) // win

    fs = jnp.dot(papers, w_src_t_ref[...],
                 preferred_element_type=f32) + bias_eff_ref[...]

    def make_oh(c):
        s_c = jnp.minimum(base_al + c * win, nseg - win)
        rel = seg - s_c
        oh = ((rel == iota_w) & (chunk_of_row == c)).astype(f32)
        return s_c, oh

    def gather_body(c, ett):
        s_c, oh = make_oh(c)
        fd_c = jnp.dot(oh, feat_dst_ref[pl.ds(s_c, win), :],
                       preferred_element_type=f32)
        ett_c = jnp.dot(oh, et_tab_ref[pl.ds(s_c, win), :],
                        preferred_element_type=f32)

        @pl.when(c == 0)
        def _():
            fd_ref[...] = fd_c

        @pl.when(c > 0)
        def _():
            fd_ref[...] += fd_c

        return ett + ett_c

    ett = lax.fori_loop(0, nch, gather_body,
                        jnp.zeros((papers.shape[0], 4), f32))

    e = jnp.sum(_leaky(fs + fd_ref[...]) * attn_ref[...],
                axis=1, keepdims=True)

    combo = a * 2 + b
    etv = jnp.zeros_like(e)
    for c in range(4):
        etv += jnp.where(combo == c, ett[:, c:c + 1], 0.0)

    mask = jnp.where(a + b == 0, MASK_VAL, 0.0)
    ex = jnp.exp(e + etv + mask)
    wp = papers * ex

    cdims = (((0,), (0,)), ((), ()))

    def scatter_body(c, _):
        s_c, oh = make_oh(c)
        acc_ref[pl.ds(s_c, win), :] += lax.dot_general(
            oh, wp, cdims, preferred_element_type=f32)
        d_ref[pl.ds(s_c, win), :] += lax.dot_general(
            oh, ex, cdims, preferred_element_type=f32)
        return 0

    lax.fori_loop(0, nch, scatter_body, 0)

    @pl.when(i == grid_n - 1)
    def _final():
        sums = acc_ref[...] / d_ref[...]
        out_ref[...] = jnp.dot(sums, w_out_t_ref[...],
                               preferred_element_type=f32) + b_out_ref[...]


def kernel(papers, snapshots, cur_snapshot_types, segment_ids, attr_a, attr_b,
           W_src, b_src, W_dst, b_dst, W_src_t, b_src_t, W_dst_t, b_dst_t,
           W_out, b_out, attn, attn_t, bias, bias_t,
           snap_emb, attr_emb_a, attr_emb_b):
    f32 = jnp.float32
    N, D = papers.shape
    B = snapshots.shape[0]

    T = 1
    for t in range(8, 1025, 8):
        if N % t == 0:
            T = t
    grid_n = N // T
    win = min(512, B)

    seg = segment_ids.astype(jnp.int32)
    seg3 = seg.reshape(grid_n, T, 1)
    a3 = attr_a.astype(jnp.int32).reshape(grid_n, T, 1)
    b3 = attr_b.astype(jnp.int32).reshape(grid_n, T, 1)
    base = seg[::T]
    maxs = seg[T - 1::T]

    S1 = snap_emb.shape[0]
    S_pad = ((S1 + 7) // 8) * 8
    snap_emb_p = jnp.pad(snap_emb, ((0, S_pad - S1), (0, 0)))
    types2 = cur_snapshot_types.astype(jnp.int32).reshape(B, 1)

    feat_dst, et_tab = pl.pallas_call(
        _tables_kernel,
        out_shape=(jax.ShapeDtypeStruct((B, D), f32),
                   jax.ShapeDtypeStruct((B, 4), f32)),
    )(snapshots, W_dst.T, b_dst.reshape(1, D), types2,
      snap_emb_p, W_dst_t.T, b_dst_t.reshape(1, D),
      attr_emb_a, attr_emb_b, W_src_t.T, b_src_t.reshape(1, D),
      attn_t, bias_t)

    bias_eff = bias + b_src.reshape(1, D)

    main = pl.pallas_call(
        functools.partial(_main_kernel, grid_n=grid_n, win=win, nseg=B),
        grid_spec=pltpu.PrefetchScalarGridSpec(
            num_scalar_prefetch=2,
            grid=(grid_n,),
            in_specs=[
                pl.BlockSpec((T, D), lambda i, b_, m_: (i, 0)),
                pl.BlockSpec((1, T, 1), lambda i, b_, m_: (i, 0, 0)),
                pl.BlockSpec((1, T, 1), lambda i, b_, m_: (i, 0, 0)),
                pl.BlockSpec((1, T, 1), lambda i, b_, m_: (i, 0, 0)),
                pl.BlockSpec((B, D), lambda i, b_, m_: (0, 0)),
                pl.BlockSpec((B, 4), lambda i, b_, m_: (0, 0)),
                pl.BlockSpec((D, D), lambda i, b_, m_: (0, 0)),
                pl.BlockSpec((1, D), lambda i, b_, m_: (0, 0)),
                pl.BlockSpec((1, D), lambda i, b_, m_: (0, 0)),
                pl.BlockSpec((D, D), lambda i, b_, m_: (0, 0)),
                pl.BlockSpec((1, D), lambda i, b_, m_: (0, 0)),
            ],
            out_specs=pl.BlockSpec((B, D), lambda i, b_, m_: (0, 0)),
            scratch_shapes=[
                pltpu.VMEM((B, D), f32),
                pltpu.VMEM((B, 1), f32),
                pltpu.VMEM((T, D), f32),
            ],
        ),
        out_shape=jax.ShapeDtypeStruct((B, D), f32),
        compiler_params=pltpu.CompilerParams(
            dimension_semantics=("arbitrary",),
        ),
    )(base, maxs,
      papers, seg3, a3, b3,
      feat_dst, et_tab, W_src.T, bias_eff,
      attn, W_out.T, b_out.reshape(1, D))

    return (main, segment_ids)


# speedup vs baseline: 4.2219x; 1.0482x over previous
import functools

import jax
import jax.numpy as jnp
from jax import lax
from jax.experimental import pallas as pl
from jax.experimental.pallas import tpu as pltpu

MASK_VAL = -100000000.0


def _leaky(x):
    return jnp.where(x >= 0, x, 0.01 * x)


def _tables_kernel(snapshots_ref, w_dst_t_ref, b_dst_ref, types_ref,
                   snap_emb_ref, w_dstt_t_ref, b_dstt_ref,
                   ea_ref, eb_ref, w_srct_t_ref, b_srct_ref,
                   attn_t_ref, bias_t_ref,
                   feat_dst_ref, et_tab_ref):
    f32 = jnp.float32
    feat_dst = jnp.dot(snapshots_ref[...], w_dst_t_ref[...],
                       preferred_element_type=f32) + b_dst_ref[...]
    feat_dst_ref[...] = feat_dst.astype(feat_dst_ref.dtype)

    ncols = snap_emb_ref.shape[0]
    iota = lax.broadcasted_iota(jnp.int32, (1, ncols), 1)
    onehot = (types_ref[...] == iota).astype(f32)
    dst_type_h = jnp.dot(onehot, snap_emb_ref[...], preferred_element_type=f32)
    dproj = jnp.dot(dst_type_h, w_dstt_t_ref[...],
                    preferred_element_type=f32) + b_dstt_ref[...]

    ea = ea_ref[...]
    eb = eb_ref[...]
    attn_t = attn_t_ref[...]
    bias_t = bias_t_ref[...]
    cols = []
    for a in (0, 1):
        for b in (0, 1):
            t = ea[a:a + 1, :] + eb[b:b + 1, :]
            tp = jnp.dot(t, w_srct_t_ref[...],
                         preferred_element_type=f32) + b_srct_ref[...]
            col = jnp.sum(_leaky(dproj + tp + bias_t) * attn_t,
                          axis=1, keepdims=True)
            cols.append(col)
    et_tab_ref[...] = jnp.concatenate(cols, axis=1).astype(et_tab_ref.dtype)


def _main_kernel(base_ref, maxs_ref,
                 papers_ref, seg_ref, a_ref, b_ref,
                 feat_dst_ref, et_tab_ref, w_src_t_ref, bias_eff_ref,
                 attn_ref, w_out_t_ref, b_out_ref,
                 out_ref, acc_ref, d_ref, fd_ref,
                 *, grid_n, win, nseg):
    f32 = jnp.float32
    i = pl.program_id(0)

    @pl.when(i == 0)
    def _init():
        acc_ref[...] = jnp.zeros_like(acc_ref)
        d_ref[...] = jnp.zeros_like(d_ref)

    papers = papers_ref[...]
    seg = seg_ref[0]
    a = a_ref[0]
    b = b_ref[0]

    base = base_ref[i]
    base_al = (base // 16) * 16
    nch = (maxs_ref[i] - base_al) // win + 1

    iota_w = lax.broadcasted_iota(jnp.int32, (1, win), 1)
    chunk_of_row = (seg - base_al) // win

    fs = jnp.dot(papers, w_src_t_ref[...],
                 preferred_element_type=f32) + bias_eff_ref[...]

    bf16 = jnp.bfloat16

    def make_oh(c):
        s_c = jnp.minimum(base_al + c * win, nseg - win)
        rel = seg - s_c
        oh = ((rel == iota_w) & (chunk_of_row == c)).astype(bf16)
        return s_c, oh

    def gather_body(c, ett):
        s_c, oh = make_oh(c)
        fd_c = jnp.dot(oh, feat_dst_ref[pl.ds(s_c, win), :],
                       preferred_element_type=f32)
        ett_c = jnp.dot(oh, et_tab_ref[pl.ds(s_c, win), :],
                        preferred_element_type=f32)

        @pl.when(c == 0)
        def _():
            fd_ref[...] = fd_c

        @pl.when(c > 0)
        def _():
            fd_ref[...] += fd_c

        return ett + ett_c

    ett = lax.fori_loop(0, nch, gather_body,
                        jnp.zeros((papers.shape[0], 4), f32))

    e = jnp.sum(_leaky(fs + fd_ref[...]) * attn_ref[...],
                axis=1, keepdims=True)

    combo = a * 2 + b
    etv = jnp.zeros_like(e)
    for c in range(4):
        etv += jnp.where(combo == c, ett[:, c:c + 1], 0.0)

    mask = jnp.where(a + b == 0, MASK_VAL, 0.0)
    ex = jnp.exp(e + etv + mask)
    wp = (papers * ex).astype(bf16)
    ex_bf = ex.astype(bf16)

    cdims = (((0,), (0,)), ((), ()))

    def scatter_body(c, _):
        s_c, oh = make_oh(c)
        acc_ref[pl.ds(s_c, win), :] += lax.dot_general(
            oh, wp, cdims, preferred_element_type=f32)
        d_ref[pl.ds(s_c, win), :] += lax.dot_general(
            oh, ex_bf, cdims, preferred_element_type=f32)
        return 0

    lax.fori_loop(0, nch, scatter_body, 0)

    @pl.when(i == grid_n - 1)
    def _final():
        sums = acc_ref[...] / d_ref[...]
        out_ref[...] = jnp.dot(sums, w_out_t_ref[...],
                               preferred_element_type=f32) + b_out_ref[...]


def kernel(papers, snapshots, cur_snapshot_types, segment_ids, attr_a, attr_b,
           W_src, b_src, W_dst, b_dst, W_src_t, b_src_t, W_dst_t, b_dst_t,
           W_out, b_out, attn, attn_t, bias, bias_t,
           snap_emb, attr_emb_a, attr_emb_b):
    f32 = jnp.float32
    N, D = papers.shape
    B = snapshots.shape[0]

    T = 1
    for t in range(8, 1025, 8):
        if N % t == 0:
            T = t
    grid_n = N // T
    win = min(256, B)

    seg = segment_ids.astype(jnp.int32)
    seg3 = seg.reshape(grid_n, T, 1)
    a3 = attr_a.astype(jnp.int32).reshape(grid_n, T, 1)
    b3 = attr_b.astype(jnp.int32).reshape(grid_n, T, 1)
    base = seg[::T]
    maxs = seg[T - 1::T]

    S1 = snap_emb.shape[0]
    S_pad = ((S1 + 7) // 8) * 8
    snap_emb_p = jnp.pad(snap_emb, ((0, S_pad - S1), (0, 0)))
    types2 = cur_snapshot_types.astype(jnp.int32).reshape(B, 1)

    feat_dst, et_tab = pl.pallas_call(
        _tables_kernel,
        out_shape=(jax.ShapeDtypeStruct((B, D), jnp.bfloat16),
                   jax.ShapeDtypeStruct((B, 4), jnp.bfloat16)),
    )(snapshots, W_dst.T, b_dst.reshape(1, D), types2,
      snap_emb_p, W_dst_t.T, b_dst_t.reshape(1, D),
      attr_emb_a, attr_emb_b, W_src_t.T, b_src_t.reshape(1, D),
      attn_t, bias_t)

    bias_eff = bias + b_src.reshape(1, D)

    main = pl.pallas_call(
        functools.partial(_main_kernel, grid_n=grid_n, win=win, nseg=B),
        grid_spec=pltpu.PrefetchScalarGridSpec(
            num_scalar_prefetch=2,
            grid=(grid_n,),
            in_specs=[
                pl.BlockSpec((T, D), lambda i, b_, m_: (i, 0)),
                pl.BlockSpec((1, T, 1), lambda i, b_, m_: (i, 0, 0)),
                pl.BlockSpec((1, T, 1), lambda i, b_, m_: (i, 0, 0)),
                pl.BlockSpec((1, T, 1), lambda i, b_, m_: (i, 0, 0)),
                pl.BlockSpec((B, D), lambda i, b_, m_: (0, 0)),
                pl.BlockSpec((B, 4), lambda i, b_, m_: (0, 0)),
                pl.BlockSpec((D, D), lambda i, b_, m_: (0, 0)),
                pl.BlockSpec((1, D), lambda i, b_, m_: (0, 0)),
                pl.BlockSpec((1, D), lambda i, b_, m_: (0, 0)),
                pl.BlockSpec((D, D), lambda i, b_, m_: (0, 0)),
                pl.BlockSpec((1, D), lambda i, b_, m_: (0, 0)),
            ],
            out_specs=pl.BlockSpec((B, D), lambda i, b_, m_: (0, 0)),
            scratch_shapes=[
                pltpu.VMEM((B, D), f32),
                pltpu.VMEM((B, 1), f32),
                pltpu.VMEM((T, D), f32),
            ],
        ),
        out_shape=jax.ShapeDtypeStruct((B, D), f32),
        compiler_params=pltpu.CompilerParams(
            dimension_semantics=("arbitrary",),
        ),
    )(base, maxs,
      papers, seg3, a3, b3,
      feat_dst, et_tab, W_src.T, bias_eff,
      attn, W_out.T, b_out.reshape(1, D))

    return (main, segment_ids)


# speedup vs baseline: 5.7255x; 1.3561x over previous
import functools

import jax
import jax.numpy as jnp
from jax import lax
from jax.experimental import pallas as pl
from jax.experimental.pallas import tpu as pltpu

MASK_VAL = -100000000.0


def _leaky(x):
    return jnp.where(x >= 0, x, 0.01 * x)


def _tables_kernel(snapshots_ref, w_dst_t_ref, b_dst_ref, types_ref,
                   snap_emb_ref, w_dstt_t_ref, b_dstt_ref,
                   ea_ref, eb_ref, w_srct_t_ref, b_srct_ref,
                   attn_t_ref, bias_t_ref,
                   feat_dst_ref, et_tab_ref):
    f32 = jnp.float32
    nb = snapshots_ref.shape[0]
    feat_dst = jnp.dot(snapshots_ref[...], w_dst_t_ref[...],
                       preferred_element_type=f32) + b_dst_ref[...]
    feat_dst_ref[0:nb, :] = feat_dst.astype(feat_dst_ref.dtype)
    feat_dst_ref[nb:, :] = jnp.zeros_like(feat_dst_ref[nb:, :])

    ncols = snap_emb_ref.shape[0]
    iota = lax.broadcasted_iota(jnp.int32, (1, ncols), 1)
    onehot = (types_ref[...] == iota).astype(f32)
    dst_type_h = jnp.dot(onehot, snap_emb_ref[...], preferred_element_type=f32)
    dproj = jnp.dot(dst_type_h, w_dstt_t_ref[...],
                    preferred_element_type=f32) + b_dstt_ref[...]

    ea = ea_ref[...]
    eb = eb_ref[...]
    attn_t = attn_t_ref[...]
    bias_t = bias_t_ref[...]
    cols = []
    for a in (0, 1):
        for b in (0, 1):
            t = ea[a:a + 1, :] + eb[b:b + 1, :]
            tp = jnp.dot(t, w_srct_t_ref[...],
                         preferred_element_type=f32) + b_srct_ref[...]
            col = jnp.sum(_leaky(dproj + tp + bias_t) * attn_t,
                          axis=1, keepdims=True)
            cols.append(col)
    et_tab_ref[0:nb, :] = jnp.concatenate(cols, axis=1).astype(et_tab_ref.dtype)
    et_tab_ref[nb:, :] = jnp.zeros_like(et_tab_ref[nb:, :])


def _main_kernel(base_ref, maxs_ref,
                 papers_ref, seg_ref, a_ref, b_ref,
                 feat_dst_ref, et_tab_ref, w_src_t_ref, bias_eff_ref,
                 attn_col_ref, w_out_t_ref, b_out_ref,
                 out_ref, acc_ref, d_ref, fd_ref,
                 *, grid_n, win, nseg):
    f32 = jnp.float32
    bf16 = jnp.bfloat16
    i = pl.program_id(0)

    @pl.when(i == 0)
    def _init():
        acc_ref[...] = jnp.zeros_like(acc_ref)
        d_ref[...] = jnp.zeros_like(d_ref)

    papers = papers_ref[...]
    seg = seg_ref[0]
    a = a_ref[0]
    b = b_ref[0]

    base_al = (base_ref[i] // 16) * 16
    nch = (maxs_ref[i] - base_al) // win + 1

    iota_w = lax.broadcasted_iota(jnp.int32, (1, win), 1)

    fs = jnp.dot(papers, w_src_t_ref[...],
                 preferred_element_type=f32) + bias_eff_ref[...]
    combo = a * 2 + b
    maskcol = jnp.where(a + b == 0, MASK_VAL, 0.0)

    def softmax_weights(fd, ett):
        x = _leaky(fs + fd)
        e = jnp.dot(x, attn_col_ref[...], preferred_element_type=f32)
        etv = jnp.zeros_like(e)
        for c in range(4):
            etv += jnp.where(combo == c, ett[:, c:c + 1], 0.0)
        return jnp.exp(e + etv + maskcol)

    cdims = (((0,), (0,)), ((), ()))

    @pl.when(nch == 1)
    def _fast():
        oh = ((seg - base_al) == iota_w).astype(bf16)
        fd = jnp.dot(oh, feat_dst_ref[pl.ds(base_al, win), :],
                     preferred_element_type=f32)
        ett = jnp.dot(oh, et_tab_ref[pl.ds(base_al, win), :],
                      preferred_element_type=f32)
        ex = softmax_weights(fd, ett)
        wp = (papers * ex).astype(bf16)
        acc_ref[pl.ds(base_al, win), :] += lax.dot_general(
            oh, wp, cdims, preferred_element_type=f32)
        d_ref[pl.ds(base_al, win), :] += lax.dot_general(
            oh, ex.astype(bf16), cdims, preferred_element_type=f32)

    @pl.when(nch > 1)
    def _general():
        def make_oh(c):
            s_c = base_al + c * win
            oh = ((seg - s_c) == iota_w).astype(bf16)
            return s_c, oh

        def gather_body(c, ett):
            s_c, oh = make_oh(c)
            fd_c = jnp.dot(oh, feat_dst_ref[pl.ds(s_c, win), :],
                           preferred_element_type=f32)
            ett_c = jnp.dot(oh, et_tab_ref[pl.ds(s_c, win), :],
                            preferred_element_type=f32)

            @pl.when(c == 0)
            def _():
                fd_ref[...] = fd_c

            @pl.when(c > 0)
            def _():
                fd_ref[...] += fd_c

            return ett + ett_c

        ett = lax.fori_loop(0, nch, gather_body,
                            jnp.zeros((papers.shape[0], 4), f32))
        ex = softmax_weights(fd_ref[...], ett)
        wp = (papers * ex).astype(bf16)
        ex_bf = ex.astype(bf16)

        def scatter_body(c, _):
            s_c, oh = make_oh(c)
            acc_ref[pl.ds(s_c, win), :] += lax.dot_general(
                oh, wp, cdims, preferred_element_type=f32)
            d_ref[pl.ds(s_c, win), :] += lax.dot_general(
                oh, ex_bf, cdims, preferred_element_type=f32)
            return 0

        lax.fori_loop(0, nch, scatter_body, 0)

    @pl.when(i == grid_n - 1)
    def _final():
        sums = acc_ref[0:nseg, :] / d_ref[0:nseg, :]
        out_ref[...] = jnp.dot(sums, w_out_t_ref[...],
                               preferred_element_type=f32) + b_out_ref[...]


def kernel(papers, snapshots, cur_snapshot_types, segment_ids, attr_a, attr_b,
           W_src, b_src, W_dst, b_dst, W_src_t, b_src_t, W_dst_t, b_dst_t,
           W_out, b_out, attn, attn_t, bias, bias_t,
           snap_emb, attr_emb_a, attr_emb_b):
    f32 = jnp.float32
    N, D = papers.shape
    B = snapshots.shape[0]

    T = 1
    for t in range(8, 1025, 8):
        if N % t == 0:
            T = t
    grid_n = N // T
    win = min(256, B)
    BP = B + win

    seg = segment_ids.astype(jnp.int32)
    seg3 = seg.reshape(grid_n, T, 1)
    a3 = attr_a.astype(jnp.int32).reshape(grid_n, T, 1)
    b3 = attr_b.astype(jnp.int32).reshape(grid_n, T, 1)
    base = seg[::T]
    maxs = seg[T - 1::T]

    S1 = snap_emb.shape[0]
    S_pad = ((S1 + 7) // 8) * 8
    snap_emb_p = jnp.pad(snap_emb, ((0, S_pad - S1), (0, 0)))
    types2 = cur_snapshot_types.astype(jnp.int32).reshape(B, 1)

    feat_dst, et_tab = pl.pallas_call(
        _tables_kernel,
        out_shape=(jax.ShapeDtypeStruct((BP, D), jnp.bfloat16),
                   jax.ShapeDtypeStruct((BP, 4), jnp.bfloat16)),
    )(snapshots, W_dst.T, b_dst.reshape(1, D), types2,
      snap_emb_p, W_dst_t.T, b_dst_t.reshape(1, D),
      attr_emb_a, attr_emb_b, W_src_t.T, b_src_t.reshape(1, D),
      attn_t, bias_t)

    bias_eff = bias + b_src.reshape(1, D)

    main = pl.pallas_call(
        functools.partial(_main_kernel, grid_n=grid_n, win=win, nseg=B),
        grid_spec=pltpu.PrefetchScalarGridSpec(
            num_scalar_prefetch=2,
            grid=(grid_n,),
            in_specs=[
                pl.BlockSpec((T, D), lambda i, b_, m_: (i, 0)),
                pl.BlockSpec((1, T, 1), lambda i, b_, m_: (i, 0, 0)),
                pl.BlockSpec((1, T, 1), lambda i, b_, m_: (i, 0, 0)),
                pl.BlockSpec((1, T, 1), lambda i, b_, m_: (i, 0, 0)),
                pl.BlockSpec((BP, D), lambda i, b_, m_: (0, 0)),
                pl.BlockSpec((BP, 4), lambda i, b_, m_: (0, 0)),
                pl.BlockSpec((D, D), lambda i, b_, m_: (0, 0)),
                pl.BlockSpec((1, D), lambda i, b_, m_: (0, 0)),
                pl.BlockSpec((D, 1), lambda i, b_, m_: (0, 0)),
                pl.BlockSpec((D, D), lambda i, b_, m_: (0, 0)),
                pl.BlockSpec((1, D), lambda i, b_, m_: (0, 0)),
            ],
            out_specs=pl.BlockSpec((B, D), lambda i, b_, m_: (0, 0)),
            scratch_shapes=[
                pltpu.VMEM((BP, D), f32),
                pltpu.VMEM((BP, 1), f32),
                pltpu.VMEM((T, D), f32),
            ],
        ),
        out_shape=jax.ShapeDtypeStruct((B, D), f32),
        compiler_params=pltpu.CompilerParams(
            dimension_semantics=("arbitrary",),
        ),
    )(base, maxs,
      papers, seg3, a3, b3,
      feat_dst, et_tab, W_src.T, bias_eff,
      attn.reshape(D, 1), W_out.T, b_out.reshape(1, D))

    return (main, segment_ids)


# speedup vs baseline: 6.0268x; 1.0526x over previous
import functools

import jax
import jax.numpy as jnp
from jax import lax
from jax.experimental import pallas as pl
from jax.experimental.pallas import tpu as pltpu

MASK_VAL = -100000000.0


def _leaky(x):
    return jnp.where(x >= 0, x, 0.01 * x)


def _tables_kernel(snapshots_ref, w_dst_t_ref, b_dst_ref, types_ref,
                   snap_emb_ref, w_dstt_t_ref, b_dstt_ref,
                   ea_ref, eb_ref, w_srct_t_ref, b_srct_ref,
                   attn_t_ref, bias_t_ref,
                   feat_dst_ref, et_tab_ref):
    f32 = jnp.float32
    nb = snapshots_ref.shape[0]
    feat_dst = jnp.dot(snapshots_ref[...], w_dst_t_ref[...],
                       preferred_element_type=f32) + b_dst_ref[...]
    feat_dst_ref[0:nb, :] = feat_dst.astype(feat_dst_ref.dtype)
    feat_dst_ref[nb:, :] = jnp.zeros_like(feat_dst_ref[nb:, :])

    ncols = snap_emb_ref.shape[0]
    iota = lax.broadcasted_iota(jnp.int32, (1, ncols), 1)
    onehot = (types_ref[...] == iota).astype(f32)
    dst_type_h = jnp.dot(onehot, snap_emb_ref[...], preferred_element_type=f32)
    dproj = jnp.dot(dst_type_h, w_dstt_t_ref[...],
                    preferred_element_type=f32) + b_dstt_ref[...]

    ea = ea_ref[...]
    eb = eb_ref[...]
    attn_t = attn_t_ref[...]
    bias_t = bias_t_ref[...]
    cols = []
    for a in (0, 1):
        for b in (0, 1):
            t = ea[a:a + 1, :] + eb[b:b + 1, :]
            tp = jnp.dot(t, w_srct_t_ref[...],
                         preferred_element_type=f32) + b_srct_ref[...]
            col = jnp.sum(_leaky(dproj + tp + bias_t) * attn_t,
                          axis=1, keepdims=True)
            cols.append(col)
    et_tab_ref[0:nb, :] = jnp.concatenate(cols, axis=1).astype(et_tab_ref.dtype)
    et_tab_ref[nb:, :] = jnp.zeros_like(et_tab_ref[nb:, :])


def _main_kernel(base_ref, maxs_ref,
                 papers_ref, seg_ref, a_ref, b_ref,
                 feat_dst_ref, et_tab_ref, w_src_t_ref, bias_eff_ref,
                 attn_col_ref, w_out_t_ref, b_out_ref,
                 out_ref, acc_ref, d_ref, fd_ref,
                 *, grid_n, win, nseg):
    f32 = jnp.float32
    bf16 = jnp.bfloat16
    i = pl.program_id(0)

    @pl.when(i == 0)
    def _init():
        acc_ref[...] = jnp.zeros_like(acc_ref)
        d_ref[...] = jnp.zeros_like(d_ref)

    papers = papers_ref[...]
    seg = seg_ref[0]
    a = a_ref[0]
    b = b_ref[0]

    base_al = (base_ref[i] // 16) * 16
    nch = (maxs_ref[i] - base_al) // win + 1

    iota_w = lax.broadcasted_iota(jnp.int32, (1, win), 1)

    fs = jnp.dot(papers, w_src_t_ref[...],
                 preferred_element_type=f32) + bias_eff_ref[...]
    combo = a * 2 + b
    maskcol = jnp.where(a + b == 0, MASK_VAL, 0.0)

    def softmax_weights(fd, ett):
        x = _leaky(fs + fd)
        e = jnp.dot(x, attn_col_ref[...], preferred_element_type=f32)
        etv = jnp.zeros_like(e)
        for c in range(4):
            etv += jnp.where(combo == c, ett[:, c:c + 1], 0.0)
        return jnp.exp(e + etv + maskcol)

    cdims = (((0,), (0,)), ((), ()))

    @pl.when(nch == 1)
    def _fast():
        oh = ((seg - base_al) == iota_w).astype(bf16)
        fd = jnp.dot(oh, feat_dst_ref[pl.ds(base_al, win), :],
                     preferred_element_type=f32)
        ett = jnp.dot(oh, et_tab_ref[pl.ds(base_al, win), :],
                      preferred_element_type=f32)
        ex = softmax_weights(fd, ett)
        wp = (papers * ex).astype(bf16)
        acc_ref[pl.ds(base_al, win), :] += lax.dot_general(
            oh, wp, cdims, preferred_element_type=f32)
        d_ref[pl.ds(base_al, win), :] += lax.dot_general(
            oh, ex.astype(bf16), cdims, preferred_element_type=f32)

    @pl.when(nch > 1)
    def _general():
        def make_oh(c):
            s_c = base_al + c * win
            oh = ((seg - s_c) == iota_w).astype(bf16)
            return s_c, oh

        def gather_body(c, ett):
            s_c, oh = make_oh(c)
            fd_c = jnp.dot(oh, feat_dst_ref[pl.ds(s_c, win), :],
                           preferred_element_type=f32)
            ett_c = jnp.dot(oh, et_tab_ref[pl.ds(s_c, win), :],
                            preferred_element_type=f32)

            @pl.when(c == 0)
            def _():
                fd_ref[...] = fd_c

            @pl.when(c > 0)
            def _():
                fd_ref[...] += fd_c

            return ett + ett_c

        ett = lax.fori_loop(0, nch, gather_body,
                            jnp.zeros((papers.shape[0], 4), f32))
        ex = softmax_weights(fd_ref[...], ett)
        wp = (papers * ex).astype(bf16)
        ex_bf = ex.astype(bf16)

        def scatter_body(c, _):
            s_c, oh = make_oh(c)
            acc_ref[pl.ds(s_c, win), :] += lax.dot_general(
                oh, wp, cdims, preferred_element_type=f32)
            d_ref[pl.ds(s_c, win), :] += lax.dot_general(
                oh, ex_bf, cdims, preferred_element_type=f32)
            return 0

        lax.fori_loop(0, nch, scatter_body, 0)

    @pl.when(i == grid_n - 1)
    def _final():
        sums = acc_ref[0:nseg, :] / d_ref[0:nseg, :]
        out_ref[...] = jnp.dot(sums, w_out_t_ref[...],
                               preferred_element_type=f32) + b_out_ref[...]


def kernel(papers, snapshots, cur_snapshot_types, segment_ids, attr_a, attr_b,
           W_src, b_src, W_dst, b_dst, W_src_t, b_src_t, W_dst_t, b_dst_t,
           W_out, b_out, attn, attn_t, bias, bias_t,
           snap_emb, attr_emb_a, attr_emb_b):
    f32 = jnp.float32
    N, D = papers.shape
    B = snapshots.shape[0]

    T = 1
    for t in range(8, 1025, 8):
        if N % t == 0:
            T = t
    grid_n = N // T
    win = min(128, B)
    BP = B + win

    seg = segment_ids.astype(jnp.int32)
    seg3 = seg.reshape(grid_n, T, 1)
    a3 = attr_a.astype(jnp.int32).reshape(grid_n, T, 1)
    b3 = attr_b.astype(jnp.int32).reshape(grid_n, T, 1)
    base = seg[::T]
    maxs = seg[T - 1::T]

    S1 = snap_emb.shape[0]
    S_pad = ((S1 + 7) // 8) * 8
    snap_emb_p = jnp.pad(snap_emb, ((0, S_pad - S1), (0, 0)))
    types2 = cur_snapshot_types.astype(jnp.int32).reshape(B, 1)

    feat_dst, et_tab = pl.pallas_call(
        _tables_kernel,
        out_shape=(jax.ShapeDtypeStruct((BP, D), jnp.bfloat16),
                   jax.ShapeDtypeStruct((BP, 4), jnp.bfloat16)),
    )(snapshots, W_dst.T, b_dst.reshape(1, D), types2,
      snap_emb_p, W_dst_t.T, b_dst_t.reshape(1, D),
      attr_emb_a, attr_emb_b, W_src_t.T, b_src_t.reshape(1, D),
      attn_t, bias_t)

    bias_eff = bias + b_src.reshape(1, D)

    main = pl.pallas_call(
        functools.partial(_main_kernel, grid_n=grid_n, win=win, nseg=B),
        grid_spec=pltpu.PrefetchScalarGridSpec(
            num_scalar_prefetch=2,
            grid=(grid_n,),
            in_specs=[
                pl.BlockSpec((T, D), lambda i, b_, m_: (i, 0)),
                pl.BlockSpec((1, T, 1), lambda i, b_, m_: (i, 0, 0)),
                pl.BlockSpec((1, T, 1), lambda i, b_, m_: (i, 0, 0)),
                pl.BlockSpec((1, T, 1), lambda i, b_, m_: (i, 0, 0)),
                pl.BlockSpec((BP, D), lambda i, b_, m_: (0, 0)),
                pl.BlockSpec((BP, 4), lambda i, b_, m_: (0, 0)),
                pl.BlockSpec((D, D), lambda i, b_, m_: (0, 0)),
                pl.BlockSpec((1, D), lambda i, b_, m_: (0, 0)),
                pl.BlockSpec((D, 1), lambda i, b_, m_: (0, 0)),
                pl.BlockSpec((D, D), lambda i, b_, m_: (0, 0)),
                pl.BlockSpec((1, D), lambda i, b_, m_: (0, 0)),
            ],
            out_specs=pl.BlockSpec((B, D), lambda i, b_, m_: (0, 0)),
            scratch_shapes=[
                pltpu.VMEM((BP, D), f32),
                pltpu.VMEM((BP, 1), f32),
                pltpu.VMEM((T, D), f32),
            ],
        ),
        out_shape=jax.ShapeDtypeStruct((B, D), f32),
        compiler_params=pltpu.CompilerParams(
            dimension_semantics=("arbitrary",),
        ),
    )(base, maxs,
      papers, seg3, a3, b3,
      feat_dst, et_tab, W_src.T, bias_eff,
      attn.reshape(D, 1), W_out.T, b_out.reshape(1, D))

    return (main, segment_ids)


# speedup vs baseline: 8.5077x; 1.4116x over previous
import dataclasses
import functools

import jax
import jax.numpy as jnp
from jax import lax
from jax.experimental import pallas as pl
from jax.experimental.pallas import tpu as pltpu
from jax.experimental.pallas import tpu_sc as plsc

MASK_VAL = -100000000.0
SC_LANES = 16
SC_WORKERS = 32


def _leaky(x):
    return jnp.where(x >= 0, x, 0.01 * x)


def _tables_kernel(snapshots_ref, w_dst_t_ref, b_dst_ref, types_ref,
                   snap_emb_ref, w_dstt_t_ref, b_dstt_ref,
                   ea_ref, eb_ref, w_srct_t_ref, b_srct_ref,
                   attn_t_ref, bias_t_ref,
                   feat_dst_ref, et4_ref):
    f32 = jnp.float32
    nb = snapshots_ref.shape[0]
    feat_dst = jnp.dot(snapshots_ref[...], w_dst_t_ref[...],
                       preferred_element_type=f32) + b_dst_ref[...]
    feat_dst_ref[0:nb, :] = feat_dst.astype(feat_dst_ref.dtype)
    feat_dst_ref[nb:, :] = jnp.zeros_like(feat_dst_ref[nb:, :])

    ncols = snap_emb_ref.shape[0]
    iota = lax.broadcasted_iota(jnp.int32, (1, ncols), 1)
    onehot = (types_ref[...] == iota).astype(f32)
    dst_type_h = jnp.dot(onehot, snap_emb_ref[...], preferred_element_type=f32)
    dproj = jnp.dot(dst_type_h, w_dstt_t_ref[...],
                    preferred_element_type=f32) + b_dstt_ref[...]

    ea = ea_ref[...]
    eb = eb_ref[...]
    attn_t = attn_t_ref[...]
    bias_t = bias_t_ref[...]
    cols = []
    for a in (0, 1):
        for b in (0, 1):
            t = ea[a:a + 1, :] + eb[b:b + 1, :]
            tp = jnp.dot(t, w_srct_t_ref[...],
                         preferred_element_type=f32) + b_srct_ref[...]
            col = jnp.sum(_leaky(dproj + tp + bias_t) * attn_t,
                          axis=1, keepdims=True)
            cols.append(col)
    et4_ref[...] = jnp.concatenate(cols, axis=1)


SC_BLK = 448


def _sc_etm_kernel(tab_hbm, code_hbm, out_hbm,
                   tab_v, code_v, out_v, sem, *, chunk):
    wid = lax.axis_index("s") * 2 + lax.axis_index("c")
    base = wid * chunk
    pltpu.async_copy(tab_hbm, tab_v, sem).wait()

    @pl.loop(0, chunk, step=SC_BLK)
    def _(off):
        pltpu.async_copy(code_hbm.at[pl.ds(base + off, SC_BLK)],
                         code_v, sem).wait()

        @pl.loop(0, SC_BLK, step=SC_LANES)
        def _(i):
            code16 = code_v[pl.ds(i, SC_LANES)]
            s16 = lax.shift_right_logical(code16, 2)
            c16 = lax.bitwise_and(code16, 3)
            v = plsc.load_gather(tab_v, [c16, s16])
            out_v[pl.ds(i, SC_LANES)] = v + jnp.where(c16 == 0, MASK_VAL, 0.0)

        pltpu.async_copy(out_v, out_hbm.at[pl.ds(base + off, SC_BLK)],
                         sem).wait()


def _sc_etm(et4, code, n):
    per_w = SC_WORKERS * SC_BLK
    npad = ((n + per_w - 1) // per_w) * per_w
    chunk = npad // SC_WORKERS
    codep = jnp.pad(code, (0, npad - n))
    mesh = plsc.VectorSubcoreMesh(core_axis_name="c", subcore_axis_name="s")
    cp = pltpu.CompilerParams()
    if "needs_layout_passes" in pltpu.CompilerParams.__dataclass_fields__:
        cp = dataclasses.replace(cp, needs_layout_passes=False)
    k = pl.kernel(
        functools.partial(_sc_etm_kernel, chunk=chunk),
        out_type=jax.ShapeDtypeStruct((npad,), jnp.float32),
        mesh=mesh,
        compiler_params=cp,
        scratch_types=[
            pltpu.VMEM(et4.shape, jnp.float32),
            pltpu.VMEM((SC_BLK,), jnp.int32),
            pltpu.VMEM((SC_BLK,), jnp.float32),
            pltpu.SemaphoreType.DMA,
        ],
    )
    return k(et4, codep)[:n]


def _main_kernel(base_ref, maxs_ref,
                 papers_ref, seg_ref, etm_ref,
                 feat_dst_ref, w_src_t_ref, bias_eff_ref,
                 attn_col_ref, w_out_t_ref, b_out_ref,
                 out_ref, acc_ref, d_ref, fd_ref,
                 *, grid_n, win, nseg):
    f32 = jnp.float32
    bf16 = jnp.bfloat16
    i = pl.program_id(0)

    @pl.when(i == 0)
    def _init():
        acc_ref[...] = jnp.zeros_like(acc_ref)
        d_ref[...] = jnp.zeros_like(d_ref)

    papers = papers_ref[...]
    seg = seg_ref[0]
    etm = etm_ref[0]

    base_al = (base_ref[i] // 16) * 16
    nch = (maxs_ref[i] - base_al) // win + 1

    iota_w = lax.broadcasted_iota(jnp.int32, (1, win), 1)

    fs = jnp.dot(papers, w_src_t_ref[...],
                 preferred_element_type=f32) + bias_eff_ref[...]

    def softmax_weights(fd):
        x = _leaky(fs + fd)
        e = jnp.dot(x, attn_col_ref[...], preferred_element_type=f32)
        return jnp.exp(e + etm)

    cdims = (((0,), (0,)), ((), ()))

    @pl.when(nch == 1)
    def _fast():
        oh = ((seg - base_al) == iota_w).astype(bf16)
        fd = jnp.dot(oh, feat_dst_ref[pl.ds(base_al, win), :],
                     preferred_element_type=f32)
        ex = softmax_weights(fd)
        wp = (papers * ex).astype(bf16)
        acc_ref[pl.ds(base_al, win), :] += lax.dot_general(
            oh, wp, cdims, preferred_element_type=f32)
        d_ref[pl.ds(base_al, win), :] += lax.dot_general(
            oh, ex.astype(bf16), cdims, preferred_element_type=f32)

    @pl.when(nch > 1)
    def _general():
        def make_oh(c):
            s_c = base_al + c * win
            oh = ((seg - s_c) == iota_w).astype(bf16)
            return s_c, oh

        def gather_body(c, _):
            s_c, oh = make_oh(c)
            fd_c = jnp.dot(oh, feat_dst_ref[pl.ds(s_c, win), :],
                           preferred_element_type=f32)

            @pl.when(c == 0)
            def _():
                fd_ref[...] = fd_c

            @pl.when(c > 0)
            def _():
                fd_ref[...] += fd_c

            return 0

        lax.fori_loop(0, nch, gather_body, 0)
        ex = softmax_weights(fd_ref[...])
        wp = (papers * ex).astype(bf16)
        ex_bf = ex.astype(bf16)

        def scatter_body(c, _):
            s_c, oh = make_oh(c)
            acc_ref[pl.ds(s_c, win), :] += lax.dot_general(
                oh, wp, cdims, preferred_element_type=f32)
            d_ref[pl.ds(s_c, win), :] += lax.dot_general(
                oh, ex_bf, cdims, preferred_element_type=f32)
            return 0

        lax.fori_loop(0, nch, scatter_body, 0)

    @pl.when(i == grid_n - 1)
    def _final():
        sums = acc_ref[0:nseg, :] / d_ref[0:nseg, :]
        out_ref[...] = jnp.dot(sums, w_out_t_ref[...],
                               preferred_element_type=f32) + b_out_ref[...]


def kernel(papers, snapshots, cur_snapshot_types, segment_ids, attr_a, attr_b,
           W_src, b_src, W_dst, b_dst, W_src_t, b_src_t, W_dst_t, b_dst_t,
           W_out, b_out, attn, attn_t, bias, bias_t,
           snap_emb, attr_emb_a, attr_emb_b):
    f32 = jnp.float32
    N, D = papers.shape
    B = snapshots.shape[0]

    T = 1
    for t in range(8, 1025, 8):
        if N % t == 0:
            T = t
    grid_n = N // T
    win = min(128, B)
    BP = B + win

    seg = segment_ids.astype(jnp.int32)
    seg3 = seg.reshape(grid_n, T, 1)
    combo = attr_a.astype(jnp.int32) * 2 + attr_b.astype(jnp.int32)
    base = seg[::T]
    maxs = seg[T - 1::T]

    S1 = snap_emb.shape[0]
    S_pad = ((S1 + 7) // 8) * 8
    snap_emb_p = jnp.pad(snap_emb, ((0, S_pad - S1), (0, 0)))
    types2 = cur_snapshot_types.astype(jnp.int32).reshape(B, 1)

    feat_dst, et4 = pl.pallas_call(
        _tables_kernel,
        out_shape=(jax.ShapeDtypeStruct((BP, D), jnp.bfloat16),
                   jax.ShapeDtypeStruct((B, 4), jnp.float32)),
    )(snapshots, W_dst.T, b_dst.reshape(1, D), types2,
      snap_emb_p, W_dst_t.T, b_dst_t.reshape(1, D),
      attr_emb_a, attr_emb_b, W_src_t.T, b_src_t.reshape(1, D),
      attn_t, bias_t)

    etm3 = _sc_etm(et4.T, seg * 4 + combo, N).reshape(grid_n, T, 1)

    bias_eff = bias + b_src.reshape(1, D)

    main = pl.pallas_call(
        functools.partial(_main_kernel, grid_n=grid_n, win=win, nseg=B),
        grid_spec=pltpu.PrefetchScalarGridSpec(
            num_scalar_prefetch=2,
            grid=(grid_n,),
            in_specs=[
                pl.BlockSpec((T, D), lambda i, b_, m_: (i, 0)),
                pl.BlockSpec((1, T, 1), lambda i, b_, m_: (i, 0, 0)),
                pl.BlockSpec((1, T, 1), lambda i, b_, m_: (i, 0, 0)),
                pl.BlockSpec((BP, D), lambda i, b_, m_: (0, 0)),
                pl.BlockSpec((D, D), lambda i, b_, m_: (0, 0)),
                pl.BlockSpec((1, D), lambda i, b_, m_: (0, 0)),
                pl.BlockSpec((D, 1), lambda i, b_, m_: (0, 0)),
                pl.BlockSpec((D, D), lambda i, b_, m_: (0, 0)),
                pl.BlockSpec((1, D), lambda i, b_, m_: (0, 0)),
            ],
            out_specs=pl.BlockSpec((B, D), lambda i, b_, m_: (0, 0)),
            scratch_shapes=[
                pltpu.VMEM((BP, D), f32),
                pltpu.VMEM((BP, 1), f32),
                pltpu.VMEM((T, D), f32),
            ],
        ),
        out_shape=jax.ShapeDtypeStruct((B, D), f32),
        compiler_params=pltpu.CompilerParams(
            dimension_semantics=("arbitrary",),
        ),
    )(base, maxs,
      papers, seg3, etm3,
      feat_dst, W_src.T, bias_eff,
      attn.reshape(D, 1), W_out.T, b_out.reshape(1, D))

    return (main, segment_ids)


# speedup vs baseline: 8.5223x; 1.0017x over previous
import dataclasses
import functools

import jax
import jax.numpy as jnp
from jax import lax
from jax.experimental import pallas as pl
from jax.experimental.pallas import tpu as pltpu
from jax.experimental.pallas import tpu_sc as plsc

MASK_VAL = -100000000.0
SC_LANES = 16
SC_WORKERS = 32


def _leaky(x):
    return jnp.where(x >= 0, x, 0.01 * x)


def _tables_kernel(snapshots_ref, w_dst_t_ref, b_dst_ref, types_ref,
                   snap_emb_ref, w_dstt_t_ref, b_dstt_ref,
                   ea_ref, eb_ref, w_srct_t_ref, b_srct_ref,
                   attn_t_ref, bias_t_ref,
                   feat_dst_ref, et4_ref):
    f32 = jnp.float32
    nb = snapshots_ref.shape[0]
    feat_dst = jnp.dot(snapshots_ref[...], w_dst_t_ref[...],
                       preferred_element_type=f32) + b_dst_ref[...]
    feat_dst_ref[0:nb, :] = feat_dst.astype(feat_dst_ref.dtype)
    feat_dst_ref[nb:, :] = jnp.zeros_like(feat_dst_ref[nb:, :])

    ncols = snap_emb_ref.shape[0]
    iota = lax.broadcasted_iota(jnp.int32, (1, ncols), 1)
    onehot = (types_ref[...] == iota).astype(f32)
    dst_type_h = jnp.dot(onehot, snap_emb_ref[...], preferred_element_type=f32)
    dproj = jnp.dot(dst_type_h, w_dstt_t_ref[...],
                    preferred_element_type=f32) + b_dstt_ref[...]

    ea = ea_ref[...]
    eb = eb_ref[...]
    attn_t = attn_t_ref[...]
    bias_t = bias_t_ref[...]
    cols = []
    for a in (0, 1):
        for b in (0, 1):
            t = ea[a:a + 1, :] + eb[b:b + 1, :]
            tp = jnp.dot(t, w_srct_t_ref[...],
                         preferred_element_type=f32) + b_srct_ref[...]
            col = jnp.sum(_leaky(dproj + tp + bias_t) * attn_t,
                          axis=1, keepdims=True)
            cols.append(col)
    et4_ref[...] = jnp.concatenate(cols, axis=1)


SC_BLK = 448


def _sc_etm_kernel(tab_hbm, code_hbm, out_hbm,
                   tab_v, code_v, out_v, sem, *, chunk):
    wid = lax.axis_index("s") * 2 + lax.axis_index("c")
    base = wid * chunk
    pltpu.async_copy(tab_hbm, tab_v, sem).wait()

    @pl.loop(0, chunk, step=SC_BLK)
    def _(off):
        pltpu.async_copy(code_hbm.at[pl.ds(base + off, SC_BLK)],
                         code_v, sem).wait()

        @pl.loop(0, SC_BLK, step=SC_LANES)
        def _(i):
            code16 = code_v[pl.ds(i, SC_LANES)]
            s16 = lax.shift_right_logical(code16, 2)
            c16 = lax.bitwise_and(code16, 3)
            v = plsc.load_gather(tab_v, [c16, s16])
            out_v[pl.ds(i, SC_LANES)] = v + jnp.where(c16 == 0, MASK_VAL, 0.0)

        pltpu.async_copy(out_v, out_hbm.at[pl.ds(base + off, SC_BLK)],
                         sem).wait()


def _sc_etm(et4, code, n):
    per_w = SC_WORKERS * SC_BLK
    npad = ((n + per_w - 1) // per_w) * per_w
    chunk = npad // SC_WORKERS
    codep = jnp.pad(code, (0, npad - n))
    mesh = plsc.VectorSubcoreMesh(core_axis_name="c", subcore_axis_name="s")
    cp = pltpu.CompilerParams()
    if "needs_layout_passes" in pltpu.CompilerParams.__dataclass_fields__:
        cp = dataclasses.replace(cp, needs_layout_passes=False)
    k = pl.kernel(
        functools.partial(_sc_etm_kernel, chunk=chunk),
        out_type=jax.ShapeDtypeStruct((npad,), jnp.float32),
        mesh=mesh,
        compiler_params=cp,
        scratch_types=[
            pltpu.VMEM(et4.shape, jnp.float32),
            pltpu.VMEM((SC_BLK,), jnp.int32),
            pltpu.VMEM((SC_BLK,), jnp.float32),
            pltpu.SemaphoreType.DMA,
        ],
    )
    return k(et4, codep)[:n]


def _main_kernel(base_ref, maxs_ref,
                 papers_ref, seg_ref, segr_ref, etm_ref,
                 feat_dst_ref, w_src_t_ref, bias_eff_ref,
                 attn_col_ref, w_out_t_ref, b_out_ref,
                 out_ref, acc_ref, d_ref, fd_ref,
                 *, grid_n, win, nseg):
    f32 = jnp.float32
    bf16 = jnp.bfloat16
    i = pl.program_id(0)

    @pl.when(i == 0)
    def _init():
        acc_ref[...] = jnp.zeros_like(acc_ref)
        d_ref[...] = jnp.zeros_like(d_ref)

    papers = papers_ref[...]
    seg = seg_ref[0]
    seg_row = segr_ref[0]
    etm = etm_ref[0]

    base_al = (base_ref[i] // 16) * 16
    nch = (maxs_ref[i] - base_al) // win + 1

    iota_w = lax.broadcasted_iota(jnp.int32, (1, win), 1)

    fs = jnp.dot(papers, w_src_t_ref[...],
                 preferred_element_type=f32) + bias_eff_ref[...]

    def softmax_weights(fd):
        x = _leaky(fs + fd)
        e = jnp.dot(x, attn_col_ref[...], preferred_element_type=f32)
        return jnp.exp(e + etm)

    cdims = (((0,), (0,)), ((), ()))

    @pl.when(nch == 1)
    def _fast():
        iota_col = lax.broadcasted_iota(jnp.int32, (win, papers.shape[0]), 0)
        oht = ((seg_row - base_al) == iota_col).astype(bf16)
        fd = lax.dot_general(oht, feat_dst_ref[pl.ds(base_al, win), :],
                             cdims, preferred_element_type=f32)
        ex = softmax_weights(fd)
        wp = (papers * ex).astype(bf16)
        acc_ref[pl.ds(base_al, win), :] += jnp.dot(
            oht, wp, preferred_element_type=f32)
        d_ref[pl.ds(base_al, win), :] += jnp.dot(
            oht, ex.astype(bf16), preferred_element_type=f32)

    @pl.when(nch > 1)
    def _general():
        def make_oh(c):
            s_c = base_al + c * win
            oh = ((seg - s_c) == iota_w).astype(bf16)
            return s_c, oh

        def gather_body(c, _):
            s_c, oh = make_oh(c)
            fd_c = jnp.dot(oh, feat_dst_ref[pl.ds(s_c, win), :],
                           preferred_element_type=f32)

            @pl.when(c == 0)
            def _():
                fd_ref[...] = fd_c

            @pl.when(c > 0)
            def _():
                fd_ref[...] += fd_c

            return 0

        lax.fori_loop(0, nch, gather_body, 0)
        ex = softmax_weights(fd_ref[...])
        wp = (papers * ex).astype(bf16)
        ex_bf = ex.astype(bf16)

        def scatter_body(c, _):
            s_c, oh = make_oh(c)
            acc_ref[pl.ds(s_c, win), :] += lax.dot_general(
                oh, wp, cdims, preferred_element_type=f32)
            d_ref[pl.ds(s_c, win), :] += lax.dot_general(
                oh, ex_bf, cdims, preferred_element_type=f32)
            return 0

        lax.fori_loop(0, nch, scatter_body, 0)

    @pl.when(i == grid_n - 1)
    def _final():
        sums = acc_ref[0:nseg, :] / d_ref[0:nseg, :]
        out_ref[...] = jnp.dot(sums, w_out_t_ref[...],
                               preferred_element_type=f32) + b_out_ref[...]


def kernel(papers, snapshots, cur_snapshot_types, segment_ids, attr_a, attr_b,
           W_src, b_src, W_dst, b_dst, W_src_t, b_src_t, W_dst_t, b_dst_t,
           W_out, b_out, attn, attn_t, bias, bias_t,
           snap_emb, attr_emb_a, attr_emb_b):
    f32 = jnp.float32
    N, D = papers.shape
    B = snapshots.shape[0]

    T = 1
    for t in range(8, 1025, 8):
        if N % t == 0:
            T = t
    grid_n = N // T
    win = min(128, B)
    BP = B + win

    seg = segment_ids.astype(jnp.int32)
    seg3 = seg.reshape(grid_n, T, 1)
    segr3 = seg.reshape(grid_n, 1, T)
    combo = attr_a.astype(jnp.int32) * 2 + attr_b.astype(jnp.int32)
    base = seg[::T]
    maxs = seg[T - 1::T]

    S1 = snap_emb.shape[0]
    S_pad = ((S1 + 7) // 8) * 8
    snap_emb_p = jnp.pad(snap_emb, ((0, S_pad - S1), (0, 0)))
    types2 = cur_snapshot_types.astype(jnp.int32).reshape(B, 1)

    feat_dst, et4 = pl.pallas_call(
        _tables_kernel,
        out_shape=(jax.ShapeDtypeStruct((BP, D), jnp.bfloat16),
                   jax.ShapeDtypeStruct((B, 4), jnp.float32)),
    )(snapshots, W_dst.T, b_dst.reshape(1, D), types2,
      snap_emb_p, W_dst_t.T, b_dst_t.reshape(1, D),
      attr_emb_a, attr_emb_b, W_src_t.T, b_src_t.reshape(1, D),
      attn_t, bias_t)

    etm3 = _sc_etm(et4.T, seg * 4 + combo, N).reshape(grid_n, T, 1)

    bias_eff = bias + b_src.reshape(1, D)

    main = pl.pallas_call(
        functools.partial(_main_kernel, grid_n=grid_n, win=win, nseg=B),
        grid_spec=pltpu.PrefetchScalarGridSpec(
            num_scalar_prefetch=2,
            grid=(grid_n,),
            in_specs=[
                pl.BlockSpec((T, D), lambda i, b_, m_: (i, 0)),
                pl.BlockSpec((1, T, 1), lambda i, b_, m_: (i, 0, 0)),
                pl.BlockSpec((1, 1, T), lambda i, b_, m_: (i, 0, 0)),
                pl.BlockSpec((1, T, 1), lambda i, b_, m_: (i, 0, 0)),
                pl.BlockSpec((BP, D), lambda i, b_, m_: (0, 0)),
                pl.BlockSpec((D, D), lambda i, b_, m_: (0, 0)),
                pl.BlockSpec((1, D), lambda i, b_, m_: (0, 0)),
                pl.BlockSpec((D, 1), lambda i, b_, m_: (0, 0)),
                pl.BlockSpec((D, D), lambda i, b_, m_: (0, 0)),
                pl.BlockSpec((1, D), lambda i, b_, m_: (0, 0)),
            ],
            out_specs=pl.BlockSpec((B, D), lambda i, b_, m_: (0, 0)),
            scratch_shapes=[
                pltpu.VMEM((BP, D), f32),
                pltpu.VMEM((BP, 1), f32),
                pltpu.VMEM((T, D), f32),
            ],
        ),
        out_shape=jax.ShapeDtypeStruct((B, D), f32),
        compiler_params=pltpu.CompilerParams(
            dimension_semantics=("arbitrary",),
        ),
    )(base, maxs,
      papers, seg3, segr3, etm3,
      feat_dst, W_src.T, bias_eff,
      attn.reshape(D, 1), W_out.T, b_out.reshape(1, D))

    return (main, segment_ids)


# speedup vs baseline: 10.0870x; 1.1836x over previous
import dataclasses
import functools

import jax
import jax.numpy as jnp
from jax import lax
from jax.experimental import pallas as pl
from jax.experimental.pallas import tpu as pltpu
from jax.experimental.pallas import tpu_sc as plsc

MASK_VAL = -100000000.0
SC_LANES = 16
SC_WORKERS = 32


def _leaky(x):
    return jnp.where(x >= 0, x, 0.01 * x)


def _tables_kernel(snapshots_ref, w_dst_t_ref, b_dst_ref, types_ref,
                   snap_emb_ref, w_dstt_t_ref, b_dstt_ref,
                   ea_ref, eb_ref, w_srct_t_ref, b_srct_ref,
                   attn_t_ref, bias_t_ref,
                   feat_dst_ref, et4_ref):
    f32 = jnp.float32
    nb = snapshots_ref.shape[0]
    feat_dst = jnp.dot(snapshots_ref[...], w_dst_t_ref[...],
                       preferred_element_type=f32) + b_dst_ref[...]
    feat_dst_ref[0:nb, :] = feat_dst.astype(feat_dst_ref.dtype)
    feat_dst_ref[nb:, :] = jnp.zeros_like(feat_dst_ref[nb:, :])

    ncols = snap_emb_ref.shape[0]
    iota = lax.broadcasted_iota(jnp.int32, (1, ncols), 1)
    onehot = (types_ref[...] == iota).astype(f32)
    dst_type_h = jnp.dot(onehot, snap_emb_ref[...], preferred_element_type=f32)
    dproj = jnp.dot(dst_type_h, w_dstt_t_ref[...],
                    preferred_element_type=f32) + b_dstt_ref[...]

    ea = ea_ref[...]
    eb = eb_ref[...]
    attn_t = attn_t_ref[...]
    bias_t = bias_t_ref[...]
    cols = []
    for a in (0, 1):
        for b in (0, 1):
            t = ea[a:a + 1, :] + eb[b:b + 1, :]
            tp = jnp.dot(t, w_srct_t_ref[...],
                         preferred_element_type=f32) + b_srct_ref[...]
            col = jnp.sum(_leaky(dproj + tp + bias_t) * attn_t,
                          axis=1, keepdims=True)
            cols.append(col)
    et4_ref[...] = jnp.concatenate(cols, axis=1)


SC_BLK = 448


def _sc_etm_kernel(tab_hbm, code_hbm, out_hbm,
                   tab_v, code_v, out_v, sem, *, chunk):
    wid = lax.axis_index("s") * 2 + lax.axis_index("c")
    base = wid * chunk
    pltpu.async_copy(tab_hbm, tab_v, sem).wait()

    @pl.loop(0, chunk, step=SC_BLK)
    def _(off):
        pltpu.async_copy(code_hbm.at[pl.ds(base + off, SC_BLK)],
                         code_v, sem).wait()

        @pl.loop(0, SC_BLK, step=SC_LANES)
        def _(i):
            code16 = code_v[pl.ds(i, SC_LANES)]
            s16 = lax.shift_right_logical(code16, 2)
            c16 = lax.bitwise_and(code16, 3)
            v = plsc.load_gather(tab_v, [c16, s16])
            out_v[pl.ds(i, SC_LANES)] = v + jnp.where(c16 == 0, MASK_VAL, 0.0)

        pltpu.async_copy(out_v, out_hbm.at[pl.ds(base + off, SC_BLK)],
                         sem).wait()


def _sc_etm(et4, code, n):
    per_w = SC_WORKERS * SC_BLK
    npad = ((n + per_w - 1) // per_w) * per_w
    chunk = npad // SC_WORKERS
    codep = jnp.pad(code, (0, npad - n))
    mesh = plsc.VectorSubcoreMesh(core_axis_name="c", subcore_axis_name="s")
    cp = pltpu.CompilerParams()
    if "needs_layout_passes" in pltpu.CompilerParams.__dataclass_fields__:
        cp = dataclasses.replace(cp, needs_layout_passes=False)
    k = pl.kernel(
        functools.partial(_sc_etm_kernel, chunk=chunk),
        out_type=jax.ShapeDtypeStruct((npad,), jnp.float32),
        mesh=mesh,
        compiler_params=cp,
        scratch_types=[
            pltpu.VMEM(et4.shape, jnp.float32),
            pltpu.VMEM((SC_BLK,), jnp.int32),
            pltpu.VMEM((SC_BLK,), jnp.float32),
            pltpu.SemaphoreType.DMA,
        ],
    )
    return k(et4, codep)[:n]


def _main_kernel(base_ref, maxs_ref,
                 papers_ref, seg_ref, segr_ref, etm_ref,
                 feat_dst_ref, w_src_t_ref, bias_eff_ref,
                 attn_col_ref, w_out_t_ref, b_out_ref,
                 out_ref, acc_ref, d_ref, fd_ref,
                 *, grid_n, win, nseg):
    f32 = jnp.float32
    bf16 = jnp.bfloat16
    i = pl.program_id(0)

    @pl.when(i == 0)
    def _init():
        acc_ref[...] = jnp.zeros_like(acc_ref)
        d_ref[...] = jnp.zeros_like(d_ref)

    papers = papers_ref[...]
    seg = seg_ref[0]
    seg_row = segr_ref[0]
    etm = etm_ref[0]

    base_al = (base_ref[i] // 16) * 16
    nch = (maxs_ref[i] - base_al) // win + 1

    iota_w = lax.broadcasted_iota(jnp.int32, (1, win), 1)

    papers_bf = papers.astype(bf16)
    fs = jnp.dot(papers_bf, w_src_t_ref[...],
                 preferred_element_type=f32) + bias_eff_ref[...]

    def softmax_weights(fd):
        x = _leaky(fs + fd)
        e = jnp.dot(x, attn_col_ref[...], preferred_element_type=f32)
        return jnp.exp(e + etm)

    cdims = (((0,), (0,)), ((), ()))

    @pl.when(nch == 1)
    def _fast():
        iota_col = lax.broadcasted_iota(jnp.int32, (win, papers.shape[0]), 0)
        oht = ((seg_row - base_al) == iota_col).astype(bf16)
        fd = lax.dot_general(oht, feat_dst_ref[pl.ds(base_al, win), :],
                             cdims, preferred_element_type=f32)
        ex = softmax_weights(fd)
        wp = (papers * ex).astype(bf16)
        acc_ref[pl.ds(base_al, win), :] += jnp.dot(
            oht, wp, preferred_element_type=f32)
        d_ref[pl.ds(base_al, win), :] += jnp.dot(
            oht, ex.astype(bf16), preferred_element_type=f32)

    @pl.when(nch > 1)
    def _general():
        def make_oh(c):
            s_c = base_al + c * win
            oh = ((seg - s_c) == iota_w).astype(bf16)
            return s_c, oh

        def gather_body(c, _):
            s_c, oh = make_oh(c)
            fd_c = jnp.dot(oh, feat_dst_ref[pl.ds(s_c, win), :],
                           preferred_element_type=f32)

            @pl.when(c == 0)
            def _():
                fd_ref[...] = fd_c

            @pl.when(c > 0)
            def _():
                fd_ref[...] += fd_c

            return 0

        lax.fori_loop(0, nch, gather_body, 0)
        ex = softmax_weights(fd_ref[...])
        wp = (papers * ex).astype(bf16)
        ex_bf = ex.astype(bf16)

        def scatter_body(c, _):
            s_c, oh = make_oh(c)
            acc_ref[pl.ds(s_c, win), :] += lax.dot_general(
                oh, wp, cdims, preferred_element_type=f32)
            d_ref[pl.ds(s_c, win), :] += lax.dot_general(
                oh, ex_bf, cdims, preferred_element_type=f32)
            return 0

        lax.fori_loop(0, nch, scatter_body, 0)

    @pl.when(i == grid_n - 1)
    def _final():
        sums = acc_ref[0:nseg, :] / d_ref[0:nseg, :]
        out_ref[...] = jnp.dot(sums, w_out_t_ref[...],
                               preferred_element_type=f32) + b_out_ref[...]


def kernel(papers, snapshots, cur_snapshot_types, segment_ids, attr_a, attr_b,
           W_src, b_src, W_dst, b_dst, W_src_t, b_src_t, W_dst_t, b_dst_t,
           W_out, b_out, attn, attn_t, bias, bias_t,
           snap_emb, attr_emb_a, attr_emb_b):
    f32 = jnp.float32
    N, D = papers.shape
    B = snapshots.shape[0]

    T = 1
    for t in range(8, 2049, 8):
        if N % t == 0:
            T = t
    grid_n = N // T
    win = min(128, B)
    BP = B + win

    seg = segment_ids.astype(jnp.int32)
    seg3 = seg.reshape(grid_n, T, 1)
    segr3 = seg.reshape(grid_n, 1, T)
    combo = attr_a.astype(jnp.int32) * 2 + attr_b.astype(jnp.int32)
    base = seg[::T]
    maxs = seg[T - 1::T]

    S1 = snap_emb.shape[0]
    S_pad = ((S1 + 7) // 8) * 8
    snap_emb_p = jnp.pad(snap_emb, ((0, S_pad - S1), (0, 0)))
    types2 = cur_snapshot_types.astype(jnp.int32).reshape(B, 1)

    feat_dst, et4 = pl.pallas_call(
        _tables_kernel,
        out_shape=(jax.ShapeDtypeStruct((BP, D), jnp.bfloat16),
                   jax.ShapeDtypeStruct((B, 4), jnp.float32)),
    )(snapshots, W_dst.T, b_dst.reshape(1, D), types2,
      snap_emb_p, W_dst_t.T, b_dst_t.reshape(1, D),
      attr_emb_a, attr_emb_b, W_src_t.T, b_src_t.reshape(1, D),
      attn_t, bias_t)

    etm3 = _sc_etm(et4.T, seg * 4 + combo, N).reshape(grid_n, T, 1)

    bias_eff = bias + b_src.reshape(1, D)

    main = pl.pallas_call(
        functools.partial(_main_kernel, grid_n=grid_n, win=win, nseg=B),
        grid_spec=pltpu.PrefetchScalarGridSpec(
            num_scalar_prefetch=2,
            grid=(grid_n,),
            in_specs=[
                pl.BlockSpec((T, D), lambda i, b_, m_: (i, 0)),
                pl.BlockSpec((1, T, 1), lambda i, b_, m_: (i, 0, 0)),
                pl.BlockSpec((1, 1, T), lambda i, b_, m_: (i, 0, 0)),
                pl.BlockSpec((1, T, 1), lambda i, b_, m_: (i, 0, 0)),
                pl.BlockSpec((BP, D), lambda i, b_, m_: (0, 0)),
                pl.BlockSpec((D, D), lambda i, b_, m_: (0, 0)),
                pl.BlockSpec((1, D), lambda i, b_, m_: (0, 0)),
                pl.BlockSpec((D, 1), lambda i, b_, m_: (0, 0)),
                pl.BlockSpec((D, D), lambda i, b_, m_: (0, 0)),
                pl.BlockSpec((1, D), lambda i, b_, m_: (0, 0)),
            ],
            out_specs=pl.BlockSpec((B, D), lambda i, b_, m_: (0, 0)),
            scratch_shapes=[
                pltpu.VMEM((BP, D), f32),
                pltpu.VMEM((BP, 1), f32),
                pltpu.VMEM((T, D), f32),
            ],
        ),
        out_shape=jax.ShapeDtypeStruct((B, D), f32),
        compiler_params=pltpu.CompilerParams(
            dimension_semantics=("arbitrary",),
        ),
    )(base, maxs,
      papers, seg3, segr3, etm3,
      feat_dst, W_src.T.astype(jnp.bfloat16), bias_eff,
      attn.reshape(D, 1), W_out.T, b_out.reshape(1, D))

    return (main, segment_ids)


# speedup vs baseline: 10.0897x; 1.0003x over previous
import dataclasses
import functools

import jax
import jax.numpy as jnp
from jax import lax
from jax.experimental import pallas as pl
from jax.experimental.pallas import tpu as pltpu
from jax.experimental.pallas import tpu_sc as plsc

MASK_VAL = -100000000.0
SC_LANES = 16
SC_WORKERS = 32


def _leaky(x):
    return jnp.where(x >= 0, x, 0.01 * x)


def _tables_kernel(snapshots_ref, w_dst_t_ref, b_dst_ref, types_ref,
                   snap_emb_ref, w_dstt_t_ref, b_dstt_ref,
                   ea_ref, eb_ref, w_srct_t_ref, b_srct_ref,
                   attn_t_ref, bias_t_ref,
                   feat_dst_ref, et4_ref):
    f32 = jnp.float32
    nb = snapshots_ref.shape[0]
    feat_dst = jnp.dot(snapshots_ref[...], w_dst_t_ref[...],
                       preferred_element_type=f32) + b_dst_ref[...]
    feat_dst_ref[0:nb, :] = feat_dst.astype(feat_dst_ref.dtype)
    feat_dst_ref[nb:, :] = jnp.zeros_like(feat_dst_ref[nb:, :])

    ncols = snap_emb_ref.shape[0]
    iota = lax.broadcasted_iota(jnp.int32, (1, ncols), 1)
    onehot = (types_ref[...] == iota).astype(f32)
    dst_type_h = jnp.dot(onehot, snap_emb_ref[...], preferred_element_type=f32)
    dproj = jnp.dot(dst_type_h, w_dstt_t_ref[...],
                    preferred_element_type=f32) + b_dstt_ref[...]

    ea = ea_ref[...]
    eb = eb_ref[...]
    attn_t = attn_t_ref[...]
    bias_t = bias_t_ref[...]
    cols = []
    for a in (0, 1):
        for b in (0, 1):
            t = ea[a:a + 1, :] + eb[b:b + 1, :]
            tp = jnp.dot(t, w_srct_t_ref[...],
                         preferred_element_type=f32) + b_srct_ref[...]
            col = jnp.sum(_leaky(dproj + tp + bias_t) * attn_t,
                          axis=1, keepdims=True)
            cols.append(col)
    et4_ref[...] = jnp.concatenate(cols, axis=1)


SC_BLK = 448


def _sc_etm_kernel(tab_hbm, code_hbm, out_hbm,
                   tab_v, code_v, out_v, sem, *, chunk):
    wid = lax.axis_index("s") * 2 + lax.axis_index("c")
    base = wid * chunk
    pltpu.async_copy(tab_hbm, tab_v, sem).wait()

    @pl.loop(0, chunk, step=SC_BLK)
    def _(off):
        pltpu.async_copy(code_hbm.at[pl.ds(base + off, SC_BLK)],
                         code_v, sem).wait()

        @pl.loop(0, SC_BLK, step=SC_LANES)
        def _(i):
            code16 = code_v[pl.ds(i, SC_LANES)]
            s16 = lax.shift_right_logical(code16, 2)
            c16 = lax.bitwise_and(code16, 3)
            v = plsc.load_gather(tab_v, [c16, s16])
            out_v[pl.ds(i, SC_LANES)] = v + jnp.where(c16 == 0, MASK_VAL, 0.0)

        pltpu.async_copy(out_v, out_hbm.at[pl.ds(base + off, SC_BLK)],
                         sem).wait()


def _sc_etm(et4, code, n):
    per_w = SC_WORKERS * SC_BLK
    npad = ((n + per_w - 1) // per_w) * per_w
    chunk = npad // SC_WORKERS
    codep = jnp.pad(code, (0, npad - n))
    mesh = plsc.VectorSubcoreMesh(core_axis_name="c", subcore_axis_name="s")
    cp = pltpu.CompilerParams()
    if "needs_layout_passes" in pltpu.CompilerParams.__dataclass_fields__:
        cp = dataclasses.replace(cp, needs_layout_passes=False)
    k = pl.kernel(
        functools.partial(_sc_etm_kernel, chunk=chunk),
        out_type=jax.ShapeDtypeStruct((npad,), jnp.float32),
        mesh=mesh,
        compiler_params=cp,
        scratch_types=[
            pltpu.VMEM(et4.shape, jnp.float32),
            pltpu.VMEM((SC_BLK,), jnp.int32),
            pltpu.VMEM((SC_BLK,), jnp.float32),
            pltpu.SemaphoreType.DMA,
        ],
    )
    return k(et4, codep)[:n]


def _main_kernel(base_ref, maxs_ref,
                 papers_ref, seg_ref, segr_ref, etm_ref,
                 feat_dst_ref, w_src_t_ref, bias_eff_ref,
                 attn_col_ref, w_out_t_ref, b_out_ref,
                 out_ref, acc_ref, d_ref, fd_ref,
                 *, grid_n, win, nseg):
    f32 = jnp.float32
    bf16 = jnp.bfloat16
    i = pl.program_id(0)

    @pl.when(i == 0)
    def _init():
        acc_ref[...] = jnp.zeros_like(acc_ref)
        d_ref[...] = jnp.zeros_like(d_ref)

    papers = papers_ref[...]
    seg = seg_ref[0]
    seg_row = segr_ref[0]
    etm = etm_ref[0]

    base_al = (base_ref[i] // 16) * 16
    nch = (maxs_ref[i] - base_al) // win + 1

    iota_w = lax.broadcasted_iota(jnp.int32, (1, win), 1)

    papers_bf = papers.astype(bf16)
    fs = jnp.dot(papers_bf, w_src_t_ref[...],
                 preferred_element_type=f32) + bias_eff_ref[...]

    def softmax_weights(fd):
        x = _leaky(fs + fd)
        e = jnp.dot(x, attn_col_ref[...], preferred_element_type=f32)
        return jnp.exp(e + etm)

    cdims = (((0,), (0,)), ((), ()))

    @pl.when(nch == 1)
    def _fast():
        iota_col = lax.broadcasted_iota(jnp.int32, (win, papers.shape[0]), 0)
        oht = ((seg_row - base_al) == iota_col).astype(bf16)
        fd = lax.dot_general(oht, feat_dst_ref[pl.ds(base_al, win), :],
                             cdims, preferred_element_type=f32)
        ex = softmax_weights(fd)
        wp = (papers * ex).astype(bf16)
        acc_ref[pl.ds(base_al, win), :] += jnp.dot(
            oht, wp, preferred_element_type=f32)
        d_ref[pl.ds(base_al, win), :] += jnp.dot(
            oht, ex.astype(bf16), preferred_element_type=f32)

    @pl.when(nch > 1)
    def _general():
        def make_oh(c):
            s_c = base_al + c * win
            oh = ((seg - s_c) == iota_w).astype(bf16)
            return s_c, oh

        def gather_body(c, _):
            s_c, oh = make_oh(c)
            fd_c = jnp.dot(oh, feat_dst_ref[pl.ds(s_c, win), :],
                           preferred_element_type=f32)

            @pl.when(c == 0)
            def _():
                fd_ref[...] = fd_c

            @pl.when(c > 0)
            def _():
                fd_ref[...] += fd_c

            return 0

        lax.fori_loop(0, nch, gather_body, 0)
        ex = softmax_weights(fd_ref[...])
        wp = (papers * ex).astype(bf16)
        ex_bf = ex.astype(bf16)

        def scatter_body(c, _):
            s_c, oh = make_oh(c)
            acc_ref[pl.ds(s_c, win), :] += lax.dot_general(
                oh, wp, cdims, preferred_element_type=f32)
            d_ref[pl.ds(s_c, win), :] += lax.dot_general(
                oh, ex_bf, cdims, preferred_element_type=f32)
            return 0

        lax.fori_loop(0, nch, scatter_body, 0)

    @pl.when(i == grid_n - 1)
    def _final():
        sums = acc_ref[0:nseg, :] / d_ref[0:nseg, :]
        out_ref[...] = jnp.dot(sums, w_out_t_ref[...],
                               preferred_element_type=f32) + b_out_ref[...]


def kernel(papers, snapshots, cur_snapshot_types, segment_ids, attr_a, attr_b,
           W_src, b_src, W_dst, b_dst, W_src_t, b_src_t, W_dst_t, b_dst_t,
           W_out, b_out, attn, attn_t, bias, bias_t,
           snap_emb, attr_emb_a, attr_emb_b):
    f32 = jnp.float32
    N, D = papers.shape
    B = snapshots.shape[0]

    T = 1
    for t in range(8, 2049, 8):
        if N % t == 0:
            T = t
    grid_n = N // T
    win = min(64, B)
    BP = B + win

    seg = segment_ids.astype(jnp.int32)
    seg3 = seg.reshape(grid_n, T, 1)
    segr3 = seg.reshape(grid_n, 1, T)
    combo = attr_a.astype(jnp.int32) * 2 + attr_b.astype(jnp.int32)
    base = seg[::T]
    maxs = seg[T - 1::T]

    S1 = snap_emb.shape[0]
    S_pad = ((S1 + 7) // 8) * 8
    snap_emb_p = jnp.pad(snap_emb, ((0, S_pad - S1), (0, 0)))
    types2 = cur_snapshot_types.astype(jnp.int32).reshape(B, 1)

    feat_dst, et4 = pl.pallas_call(
        _tables_kernel,
        out_shape=(jax.ShapeDtypeStruct((BP, D), jnp.bfloat16),
                   jax.ShapeDtypeStruct((B, 4), jnp.float32)),
    )(snapshots, W_dst.T, b_dst.reshape(1, D), types2,
      snap_emb_p, W_dst_t.T, b_dst_t.reshape(1, D),
      attr_emb_a, attr_emb_b, W_src_t.T, b_src_t.reshape(1, D),
      attn_t, bias_t)

    etm3 = _sc_etm(et4.T, seg * 4 + combo, N).reshape(grid_n, T, 1)

    bias_eff = bias + b_src.reshape(1, D)

    main = pl.pallas_call(
        functools.partial(_main_kernel, grid_n=grid_n, win=win, nseg=B),
        grid_spec=pltpu.PrefetchScalarGridSpec(
            num_scalar_prefetch=2,
            grid=(grid_n,),
            in_specs=[
                pl.BlockSpec((T, D), lambda i, b_, m_: (i, 0)),
                pl.BlockSpec((1, T, 1), lambda i, b_, m_: (i, 0, 0)),
                pl.BlockSpec((1, 1, T), lambda i, b_, m_: (i, 0, 0)),
                pl.BlockSpec((1, T, 1), lambda i, b_, m_: (i, 0, 0)),
                pl.BlockSpec((BP, D), lambda i, b_, m_: (0, 0)),
                pl.BlockSpec((D, D), lambda i, b_, m_: (0, 0)),
                pl.BlockSpec((1, D), lambda i, b_, m_: (0, 0)),
                pl.BlockSpec((D, 1), lambda i, b_, m_: (0, 0)),
                pl.BlockSpec((D, D), lambda i, b_, m_: (0, 0)),
                pl.BlockSpec((1, D), lambda i, b_, m_: (0, 0)),
            ],
            out_specs=pl.BlockSpec((B, D), lambda i, b_, m_: (0, 0)),
            scratch_shapes=[
                pltpu.VMEM((BP, D), f32),
                pltpu.VMEM((BP, 1), f32),
                pltpu.VMEM((T, D), f32),
            ],
        ),
        out_shape=jax.ShapeDtypeStruct((B, D), f32),
        compiler_params=pltpu.CompilerParams(
            dimension_semantics=("arbitrary",),
        ),
    )(base, maxs,
      papers, seg3, segr3, etm3,
      feat_dst, W_src.T.astype(jnp.bfloat16), bias_eff,
      attn.reshape(D, 1), W_out.T, b_out.reshape(1, D))

    return (main, segment_ids)
